# baseline (speedup 1.0000x reference)
import functools

import jax
import jax.numpy as jnp
from jax.experimental import pallas as pl
from jax.experimental.pallas import tpu as pltpu
from jax.experimental.pallas import tpu_sc as plsc

B = 4096
S = 200
D = 128
TOKS = B * S
GATHER_W = 128
BB = 32


def _sc_gather(tok_table, x_flat, n_rows):
    mesh = plsc.VectorSubcoreMesh(core_axis_name="c", subcore_axis_name="s")
    num_windows = n_rows // GATHER_W

    @functools.partial(
        pl.kernel,
        out_type=jax.ShapeDtypeStruct((n_rows, D), jnp.float32),
        mesh=mesh,
    )
    def gather_kernel(tok_hbm, idx_hbm, out_hbm):
        def body(idx_vmem, out_vmem):
            pltpu.sync_copy(tok_hbm.at[idx_vmem.at[0]], out_vmem)

        pltpu.emit_pipeline(
            body,
            grid=(num_windows,),
            in_specs=[pl.BlockSpec((1, GATHER_W), index_map=lambda i: (0, i))],
            out_specs=[pl.BlockSpec((GATHER_W, D), index_map=lambda i: (i, 0))],
            core_axis_name=("c", "s"),
            dimension_semantics=(pltpu.PARALLEL,),
        )(idx_hbm, out_hbm)

    return gather_kernel(tok_table, x_flat.reshape(1, n_rows))


def _ln_body(g_ref, seg_ref, pos_ref, segd_ref, gam_ref, bet_ref, o_ref):
    segb = seg_ref[...]
    h = g_ref[...] + pos_ref[...] + segb * segd_ref[...]
    ones = jnp.full((D, D), 1.0 / D, jnp.float32)
    dims = (((2,), (0,)), ((), ()))
    mu = jax.lax.dot_general(h, ones, dims)
    sq = jax.lax.dot_general(h * h, ones, dims)
    var = sq - mu * mu
    o_ref[...] = (h - mu) * jax.lax.rsqrt(var + 1e-5) * gam_ref[...] + bet_ref[...]


def kernel(x, seg, tok_table, pos_table, seg_table, ln_gamma, ln_beta):
    x_flat = x.reshape(-1).astype(jnp.int32)
    segf = seg.astype(jnp.float32).reshape(B, S, 1)
    pos3 = (pos_table[:S] + seg_table[0][None, :]).reshape(1, S, D)
    segd = (seg_table[1] - seg_table[0]).reshape(1, 1, D)
    gamma = ln_gamma.reshape(1, 1, D)
    beta = ln_beta.reshape(1, 1, D)

    gathered = _sc_gather(tok_table, x_flat, TOKS).reshape(B, S, D)
    return pl.pallas_call(
        _ln_body,
        grid=(B // BB,),
        in_specs=[
            pl.BlockSpec((BB, S, D), lambda i: (i, 0, 0)),
            pl.BlockSpec((BB, S, 1), lambda i: (i, 0, 0)),
            pl.BlockSpec((1, S, D), lambda i: (0, 0, 0)),
            pl.BlockSpec((1, 1, D), lambda i: (0, 0, 0)),
            pl.BlockSpec((1, 1, D), lambda i: (0, 0, 0)),
            pl.BlockSpec((1, 1, D), lambda i: (0, 0, 0)),
        ],
        out_specs=pl.BlockSpec((BB, S, D), lambda i: (i, 0, 0)),
        out_shape=jax.ShapeDtypeStruct((B, S, D), jnp.float32),
        compiler_params=pltpu.CompilerParams(
            dimension_semantics=("parallel",)),
    )(gathered, segf, pos3, segd, gamma, beta)
